# baseline (speedup 1.0000x reference)
import functools
import math

import jax
import jax.numpy as jnp
from jax import lax
from jax.experimental import pallas as pl
from jax.experimental.pallas import tpu as pltpu
from jax.experimental.pallas import tpu_sc as plsc

F32 = jnp.float32
_NC, _NS, _NW = 2, 16, 32


def _sc_mesh():
    return plsc.VectorSubcoreMesh(core_axis_name="c", subcore_axis_name="s",
                                  num_cores=_NC, num_subcores=_NS)


def _rup(x, m):
    return (x + m - 1) // m * m


def _mm_bt_kernel(a_ref, b_ref, o_ref, *, nk, zero_diag, bm, bn, bk,
                  pa_ref=None, pb_ref=None):
    k = pl.program_id(2)
    i, j = pl.program_id(0), pl.program_id(1)

    @pl.when(k == 0)
    def _():
        o_ref[...] = jnp.zeros_like(o_ref)

    a = a_ref[...]
    b = b_ref[...]
    if pa_ref is not None:
        kca = lax.broadcasted_iota(jnp.int32, (bm, bk), 1) + k * bk
        a = a + jnp.where(kca == pa_ref[:, 0:1], 1.0, 0.0)
        kcb = lax.broadcasted_iota(jnp.int32, (bn, bk), 1) + k * bk
        b = b + jnp.where(kcb == pb_ref[:, 0:1], 1.0, 0.0)
    if a.dtype != jnp.bfloat16:
        a = a.astype(jnp.bfloat16)
        b = b.astype(jnp.bfloat16)
    o_ref[...] += lax.dot_general(
        a, b, (((1,), (1,)), ((), ())), preferred_element_type=F32)

    if zero_diag:
        @pl.when(k == nk - 1)
        def _():
            ri = lax.broadcasted_iota(jnp.int32, (bm, bn), 0) + i * bm
            ci = lax.broadcasted_iota(jnp.int32, (bm, bn), 1) + j * bn
            o_ref[...] = jnp.where(ri == ci, 0.0, o_ref[...])


def _mm_bt_kernel_d(a_ref, b_ref, pa_ref, pb_ref, o_ref, **kw):
    _mm_bt_kernel(a_ref, b_ref, o_ref, pa_ref=pa_ref, pb_ref=pb_ref, **kw)


def _mm_bt(A, B, zero_diag=False, diagp=None, bm=256, bn=256, bk=1024):
    M, K = A.shape
    N, _ = B.shape
    bm, bn, bk = min(bm, M), min(bn, N), min(bk, K)
    nk = K // bk
    in_specs = [
        pl.BlockSpec((bm, bk), lambda i, j, k: (i, k)),
        pl.BlockSpec((bn, bk), lambda i, j, k: (j, k)),
    ]
    args = [A, B]
    if diagp is not None:
        in_specs += [
            pl.BlockSpec((bm, 128), lambda i, j, k: (i, 0)),
            pl.BlockSpec((bn, 128), lambda i, j, k: (j, 0)),
        ]
        args += [diagp, diagp]
        kfn = functools.partial(_mm_bt_kernel_d, nk=nk, zero_diag=zero_diag,
                                bm=bm, bn=bn, bk=bk)
    else:
        kfn = functools.partial(_mm_bt_kernel, nk=nk, zero_diag=zero_diag,
                                bm=bm, bn=bn, bk=bk)
    return pl.pallas_call(
        kfn,
        grid=(M // bm, N // bn, nk),
        in_specs=in_specs,
        out_specs=pl.BlockSpec((bm, bn), lambda i, j, k: (i, j)),
        out_shape=jax.ShapeDtypeStruct((M, N), F32),
    )(*args)


def _conv_kernel(st_ref, y_ref, ye_ref, cnt_ref, b_ref, o_ref, *, nk, bm, bn):
    k = pl.program_id(2)

    @pl.when(k == 0)
    def _():
        o_ref[...] = jnp.zeros_like(o_ref)

    o_ref[...] += lax.dot_general(
        st_ref[...], y_ref[...], (((1,), (0,)), ((), ())),
        preferred_element_type=F32)

    @pl.when(k == nk - 1)
    def _():
        dinv = lax.rsqrt(cnt_ref[:, 0:1] + 1.0)
        bias = jnp.broadcast_to(b_ref[...], (bm, bn))
        o_ref[...] = jnp.maximum(dinv * (o_ref[...] + ye_ref[...]) + bias, 0.0)


def _conv_apply(StT, y, cnt_b, bias, bm=512, bk=1024):
    M, K = StT.shape
    _, F = y.shape
    bn = F
    bm = min(bm, M)
    bk = min(bk, K)
    while K % bk:
        bk //= 2
    nk = K // bk
    return pl.pallas_call(
        functools.partial(_conv_kernel, nk=nk, bm=bm, bn=bn),
        grid=(M // bm, F // bn, nk),
        in_specs=[
            pl.BlockSpec((bm, bk), lambda i, j, k: (i, k)),
            pl.BlockSpec((bk, bn), lambda i, j, k: (k, j)),
            pl.BlockSpec((bm, bn), lambda i, j, k: (i, j)),
            pl.BlockSpec((bm, 128), lambda i, j, k: (i, 0)),
            pl.BlockSpec((1, bn), lambda i, j, k: (0, j)),
        ],
        out_specs=pl.BlockSpec((bm, bn), lambda i, j, k: (i, j)),
        out_shape=jax.ShapeDtypeStruct((M, F), F32),
    )(StT, y, y, cnt_b, bias)


def _xw_kernel(x_ref, w_ref, cnt_ref, o_ref):
    acc = lax.dot_general(x_ref[...], w_ref[...], (((1,), (0,)), ((), ())),
                          preferred_element_type=F32)
    dinv = lax.rsqrt(cnt_ref[:, 0:1] + 1.0)
    o_ref[...] = acc * dinv


def _xw_kernels(x_ref, sc_ref, w_ref, cnt_ref, o_ref):
    xin = x_ref[...] * sc_ref[:, 0:1]
    acc = lax.dot_general(xin, w_ref[...], (((1,), (0,)), ((), ())),
                          preferred_element_type=F32)
    dinv = lax.rsqrt(cnt_ref[:, 0:1] + 1.0)
    o_ref[...] = acc * dinv


def _xw_apply(x, W, cnt_b, scale=None, bm=512):
    M, K = x.shape
    _, F = W.shape
    bn = F
    bm = min(bm, M)
    args = [x] + ([scale] if scale is not None else []) + [W, cnt_b]
    in_specs = [pl.BlockSpec((bm, K), lambda i, j: (i, 0))]
    if scale is not None:
        in_specs.append(pl.BlockSpec((bm, 128), lambda i, j: (i, 0)))
    in_specs += [
        pl.BlockSpec((K, bn), lambda i, j: (0, j)),
        pl.BlockSpec((bm, 128), lambda i, j: (i, 0)),
    ]
    kfn = _xw_kernels if scale is not None else _xw_kernel
    return pl.pallas_call(
        kfn,
        grid=(M // bm, F // bn),
        in_specs=in_specs,
        out_specs=pl.BlockSpec((bm, bn), lambda i, j: (i, j)),
        out_shape=jax.ShapeDtypeStruct((M, F), F32),
    )(*args)


def _score_kernel(x_ref, p_ref, sb_ref, st_ref):
    pcol = p_ref[:, 0:1]
    ns = jnp.sum(pcol * pcol)
    xp = lax.dot_general(x_ref[...], p_ref[...], (((1,), (0,)), ((), ())),
                         preferred_element_type=F32)
    s = jnp.tanh(xp / jnp.sqrt(ns))
    sb_ref[...] = s
    st_ref[...] = s.T


def _scores(x, p_b, bm=512):
    M, K = x.shape
    bm = min(bm, M)
    return pl.pallas_call(
        _score_kernel,
        grid=(M // bm,),
        in_specs=[
            pl.BlockSpec((bm, K), lambda i: (i, 0)),
            pl.BlockSpec((K, 128), lambda i: (0, 0)),
        ],
        out_specs=[
            pl.BlockSpec((bm, 128), lambda i: (i, 0)),
            pl.BlockSpec((128, bm), lambda i: (0, i)),
        ],
        out_shape=[
            jax.ShapeDtypeStruct((M, 128), F32),
            jax.ShapeDtypeStruct((128, M), F32),
        ],
    )(x, p_b)


def _rank_kernel(sb_ref, st_ref, o_ref, *, bi, bj, nj, n_real):
    j = pl.program_id(1)

    @pl.when(j == 0)
    def _():
        o_ref[...] = jnp.zeros_like(o_ref)

    si = sb_ref[:, 0:1]
    sj = st_ref[0:1, :]
    ig = lax.broadcasted_iota(jnp.int32, (bi, bj), 0) + pl.program_id(0) * bi
    jg = lax.broadcasted_iota(jnp.int32, (bi, bj), 1) + j * bj
    valid = jg < n_real
    cmp = (sj > si) | ((sj == si) & (jg < ig))
    cnt = jnp.sum(jnp.where(cmp & valid, 1, 0), axis=1, keepdims=True)
    o_ref[...] += jnp.broadcast_to(cnt, o_ref.shape)

    @pl.when(j == nj - 1)
    def _():
        pad = jnp.where(ig[:, 0:1] >= n_real, jnp.int32(1 << 28), 0)
        o_ref[...] += jnp.broadcast_to(pad, o_ref.shape)


def _ranks(sb, st, n_real, bi=256, bj=512):
    M = sb.shape[0]
    bi, bj = min(bi, M), min(bj, M)
    nj = M // bj
    return pl.pallas_call(
        functools.partial(_rank_kernel, bi=bi, bj=bj, nj=nj, n_real=n_real),
        grid=(M // bi, nj),
        in_specs=[
            pl.BlockSpec((bi, 128), lambda i, j: (i, 0)),
            pl.BlockSpec((128, bj), lambda i, j: (0, j)),
        ],
        out_specs=pl.BlockSpec((bi, 128), lambda i, j: (i, 0)),
        out_shape=jax.ShapeDtypeStruct((M, 128), jnp.int32),
    )(sb, st)


def _tr_kernel(i_ref, o_ref):
    o_ref[...] = i_ref[...].T


def _transpose(A, bm=256, bn=256):
    M, N = A.shape
    bm, bn = min(bm, M), min(bn, N)
    return pl.pallas_call(
        _tr_kernel,
        grid=(M // bm, N // bn),
        in_specs=[pl.BlockSpec((bm, bn), lambda i, j: (i, j))],
        out_specs=pl.BlockSpec((bn, bm), lambda i, j: (j, i)),
        out_shape=jax.ShapeDtypeStruct((N, M), F32),
    )(A)


def _rowsum_kernel(a_ref, o_ref, *, nk):
    k = pl.program_id(1)

    @pl.when(k == 0)
    def _():
        o_ref[...] = jnp.zeros_like(o_ref)

    s = jnp.sum(a_ref[...], axis=1, keepdims=True)
    o_ref[...] += jnp.broadcast_to(s, o_ref.shape)


def _rowsum(A, bm=256, bk=2560):
    M, K = A.shape
    bm, bk = min(bm, M), min(bk, K)
    nk = K // bk
    return pl.pallas_call(
        functools.partial(_rowsum_kernel, nk=nk),
        grid=(M // bm, nk),
        in_specs=[pl.BlockSpec((bm, bk), lambda i, k: (i, k))],
        out_specs=pl.BlockSpec((bm, 128), lambda i, k: (i, 0)),
        out_shape=jax.ShapeDtypeStruct((M, 128), F32),
    )(A)


def _sc_gather_multi(tables, idx):
    B = idx.shape[0]
    nt = len(tables)
    bpw = B // _NW
    chunks = []
    for t in tables:
        D = t.shape[1]
        C = min(bpw, 128)
        while C * D * 4 > 196608 or bpw % C:
            C -= 1
        chunks.append(C)
    ipad = _rup(bpw + 16, 16)

    @functools.partial(
        pl.kernel, mesh=_sc_mesh(),
        out_type=[jax.ShapeDtypeStruct((B, t.shape[1]), t.dtype)
                  for t in tables],
        scratch_types=[pltpu.VMEM((ipad,), jnp.int32)] + [
            pltpu.VMEM((chunks[t], tables[t].shape[1]), tables[t].dtype)
            for t in range(nt)] + [pltpu.SemaphoreType.DMA])
    def k(*refs):
        tabs_h = refs[:nt]
        idx_h = refs[nt]
        outs_h = refs[nt + 1:2 * nt + 1]
        idx_v = refs[2 * nt + 1]
        rows_vs = refs[2 * nt + 2:3 * nt + 2]
        sem = refs[3 * nt + 2]
        wid = lax.axis_index("s") * _NC + lax.axis_index("c")
        base = wid * bpw
        pltpu.sync_copy(idx_h.at[pl.ds(base, bpw)], idx_v.at[pl.ds(0, bpw)])

        for t in range(nt):
            C = chunks[t]
            rows_v = rows_vs[t]

            def body(g, t=t, C=C, rows_v=rows_v):
                pltpu.async_copy(tabs_h[t].at[idx_v.at[pl.ds(g * C, C)]],
                                 rows_v, sem).wait()
                pltpu.sync_copy(rows_v, outs_h[t].at[pl.ds(base + g * C, C)])

            lax.fori_loop(0, bpw // C,
                          lambda g, _, b=body: (b(g), 0)[1], 0)

    return k(*tables, idx)


def _sc_gather_rows(table, idx):
    return _sc_gather_multi([table], idx)[0]


def _sc_topk_select(rank, s, k, kp, dump):
    M = rank.shape[0]

    @functools.partial(
        pl.kernel, mesh=_sc_mesh(),
        out_type=[jax.ShapeDtypeStruct((kp,), jnp.int32),
                  jax.ShapeDtypeStruct((kp,), F32)],
        scratch_types=[
            pltpu.VMEM((M,), jnp.int32),
            pltpu.VMEM((M,), F32),
            pltpu.VMEM((M,), jnp.int32),
            pltpu.VMEM((M // 128, 128), jnp.int32),
            pltpu.VMEM((kp + 128,), jnp.int32),
            pltpu.VMEM((kp + 128,), F32),
            pltpu.VMEM_SHARED((kp + 128,), jnp.int32),
            pltpu.VMEM_SHARED((kp + 128,), F32),
        ])
    def kk(rank_h, s_h, perm_h, vals_h, r_v, s_v, id_v, tgt_v, pi_v, vi_v,
           p_v, v_v):
        wid = lax.axis_index("s") * _NC + lax.axis_index("c")

        @pl.when(wid == 0)
        def _():
            pltpu.sync_copy(rank_h, r_v)
            pltpu.sync_copy(s_h, s_v)

            def init(i, _):
                pi_v[pl.ds(i * 16, 16)] = jnp.full((16,), dump, jnp.int32)
                vi_v[pl.ds(i * 16, 16)] = jnp.zeros((16,), F32)
                return 0

            lax.fori_loop(0, (kp + 128) // 16, init, 0)
            pltpu.sync_copy(pi_v, p_v)
            pltpu.sync_copy(vi_v, v_v)

            def prep(i, _):
                r = r_v[pl.ds(i * 16, 16)]
                tq = jnp.where(r < k, r, jnp.int32(kp))
                tgt_v[i // 8, pl.ds((i % 8) * 16, 16)] = tq
                id_v[pl.ds(i * 16, 16)] = lax.iota(jnp.int32, 16) + i * 16
                return 0

            lax.fori_loop(0, M // 16, prep, 0)

            def scat(j, _):
                pltpu.sync_copy(id_v.at[pl.ds(j * 128, 128)],
                                p_v.at[tgt_v.at[j]])
                pltpu.sync_copy(s_v.at[pl.ds(j * 128, 128)],
                                v_v.at[tgt_v.at[j]])
                return 0

            lax.fori_loop(0, M // 128, scat, 0)
            pltpu.sync_copy(p_v.at[pl.ds(0, kp)], perm_h)
            pltpu.sync_copy(v_v.at[pl.ds(0, kp)], vals_h)

    return kk(rank, s)


def _sc_up_add(res, xup, perm):
    M, F = res.shape
    B = perm.shape[0]
    bpw = B // _NS
    half = M // _NC
    rpt = M // _NW
    CC = 64
    while rpt % CC or CC * F * 4 > 131072:
        CC //= 2
    nsc = bpw // 80

    @functools.partial(
        pl.kernel, mesh=_sc_mesh(),
        out_type=jax.ShapeDtypeStruct((M + 8, F), F32),
        scratch_types=[
            pltpu.VMEM((_rup(bpw, 16),), jnp.int32),
            pltpu.VMEM((nsc, 80), jnp.int32),
            pltpu.VMEM((80, F), F32),
            pltpu.VMEM((80, F), F32),
            pltpu.SemaphoreType.DMA,
        ])
    def k(res_h, xup_h, perm_h, out_h, idx_v, tgt_v, xu_v, gat_v, sem):
        cid = lax.axis_index("c")
        sid = lax.axis_index("s")
        wid2 = cid * _NS + sid

        def cbody(t):
            st = wid2 * rpt + t * CC
            pltpu.sync_copy(res_h.at[pl.ds(st, CC)], gat_v.at[pl.ds(0, CC)])
            pltpu.sync_copy(gat_v.at[pl.ds(0, CC)], out_h.at[pl.ds(st, CC)])

        lax.fori_loop(0, rpt // CC, lambda t, _: (cbody(t), 0)[1], 0)
        plsc.subcore_barrier()

        base = sid * bpw
        pltpu.sync_copy(perm_h.at[pl.ds(base, bpw)],
                        idx_v.at[pl.ds(0, bpw)])
        lo = cid * half

        def abody(i):
            p = idx_v[pl.ds(i * 16, 16)]
            inh = (p >= lo) & (p < lo + half)
            tq = jnp.where(inh, p, jnp.int32(M))
            tgt_v[i // 5, pl.ds((i % 5) * 16, 16)] = tq

        lax.fori_loop(0, bpw // 16, lambda i, _: (abody(i), 0)[1], 0)

        nv = F // 16

        def sbody(r):
            pltpu.sync_copy(xup_h.at[pl.ds(base + r * 80, 80)], xu_v)
            pltpu.async_copy(res_h.at[idx_v.at[pl.ds(r * 80, 80)]],
                             gat_v, sem).wait()

            def inner(j, _):
                gat_v[j // nv, pl.ds((j % nv) * 16, 16)] = (
                    gat_v[j // nv, pl.ds((j % nv) * 16, 16)]
                    + xu_v[j // nv, pl.ds((j % nv) * 16, 16)])
                return 0

            lax.fori_loop(0, 80 * nv, inner, 0)
            pltpu.async_copy(gat_v, out_h.at[tgt_v.at[r]], sem).wait()

        lax.fori_loop(0, nsc, lambda r, _: (sbody(r), 0)[1], 0)

    return k(res, xup, perm)[:M]


def _sc_edge_agg(src, dst, ys, n_out):
    E = src.shape[0]
    F = 128
    nph = len(ys)
    half = n_out // _NC
    ept = E // _NS
    nch = ept // 64
    arows = half + 128
    fpt = half // _NS

    @functools.partial(
        pl.kernel, mesh=_sc_mesh(),
        out_type=[jax.ShapeDtypeStruct((n_out, F), F32) for _ in ys],
        scratch_types=[
            pltpu.VMEM((ept,), jnp.int32),
            pltpu.VMEM((ept,), jnp.int32),
            pltpu.VMEM((nch, 64), jnp.int32),
            pltpu.VMEM((64, F), F32),
            pltpu.VMEM((64, F), F32),
            pltpu.VMEM((64, F), F32),
            pltpu.SemaphoreType.DMA,
            pltpu.SemaphoreType.DMA,
            pltpu.VMEM_SHARED((arows, F), F32),
        ])
    def k(src_h, dst_h, *rest):
        ys_h = rest[:nph]
        outs_h = rest[nph:2 * nph]
        src_v, dst_v, tgt_v, zb_v, ra_v, rb_v, sema, semb, acc = (
            rest[2 * nph:])
        cid = lax.axis_index("c")
        sid = lax.axis_index("s")
        lo = cid * half

        def zrow(j, _):
            zb_v[j // 8, pl.ds((j % 8) * 16, 16)] = jnp.zeros((16,), F32)
            return 0

        lax.fori_loop(0, 64 * 8, zrow, 0)

        eb = sid * ept
        pltpu.sync_copy(src_h.at[pl.ds(eb, ept)], src_v)
        pltpu.sync_copy(dst_h.at[pl.ds(eb, ept)], dst_v)

        def tb(i):
            d = dst_v[pl.ds(i * 16, 16)]
            inh = (d >= lo) & (d < lo + half)
            tq = jnp.where(inh, d - lo, jnp.int32(half))
            tgt_v[i // 4, pl.ds((i % 4) * 16, 16)] = tq

        lax.fori_loop(0, ept // 16, lambda i, _: (tb(i), 0)[1], 0)

        zpt = arows // _NS

        for f in range(nph):
            y_h = ys_h[f]
            out_h = outs_h[f]

            def zc2(u, _):
                pltpu.sync_copy(zb_v, acc.at[pl.ds(sid * zpt + u * 64, 64)])
                return 0

            lax.fori_loop(0, zpt // 64, zc2, 0)
            pltpu.sync_copy(
                zb_v.at[pl.ds(0, zpt - (zpt // 64) * 64)],
                acc.at[pl.ds(sid * zpt + (zpt // 64) * 64,
                             zpt - (zpt // 64) * 64)])
            plsc.subcore_barrier()

            def gath(g, buf, sem):
                return pltpu.async_copy(
                    y_h.at[src_v.at[pl.ds(g * 64, 64)]], buf, sem)

            gath(0, ra_v, sema).wait()

            def pbody(g0):
                hb = gath(g0 + 1, rb_v, semb)
                pltpu.sync_copy(ra_v, acc.at[tgt_v.at[g0]], add=True)
                g2 = jnp.minimum(g0 + 2, nch - 2)
                ha = gath(g2, ra_v, sema)
                hb.wait()
                pltpu.sync_copy(rb_v, acc.at[tgt_v.at[g0 + 1]], add=True)
                ha.wait()

            lax.fori_loop(0, nch // 2, lambda t, _: (pbody(t * 2), 0)[1], 0)
            plsc.subcore_barrier()

            def fcp(u, _):
                st = sid * fpt + u * 64
                pltpu.sync_copy(acc.at[pl.ds(st, 64)],
                                out_h.at[pl.ds(lo + st, 64)])
                return 0

            lax.fori_loop(0, fpt // 64, fcp, 0)
            plsc.subcore_barrier()

    return k(src, dst, *ys)


def _scatter_counts(dst, n):
    return jnp.zeros((n,), F32).at[dst].add(1.0, mode="drop")


def _topk_select(rank, s, k, kp, dump):
    sel = rank < k
    idx = jnp.arange(rank.shape[0], dtype=jnp.int32)
    tgt = jnp.where(sel, rank, k)
    perm = jnp.full((kp + 1,), dump, jnp.int32).at[tgt].set(
        jnp.where(sel, idx, dump), mode="drop")
    vals = jnp.zeros((kp + 1,), F32).at[tgt].set(
        jnp.where(sel, s, 0.0), mode="drop")
    perm = perm.at[k].set(dump) if k < kp else perm
    vals = vals.at[k].set(0.0) if k < kp else vals
    return perm[:kp], vals[:kp]


def _build_restricted(inv, rows_idx, cols_idx, kp, n):
    a = inv[rows_idx]
    w = jnp.where(a >= 0, 1.0, 0.0).astype(jnp.bfloat16)
    a = jnp.where(a >= 0, a, kp)
    out = jnp.zeros((kp + 1, n), jnp.bfloat16).at[a, cols_idx].add(
        w, mode="drop")
    return out[:kp]


def kernel(x, edge_index, W0, b0, W1, b1, W2, b2, W3, b3, W4, b4, p0, p1):
    n0 = x.shape[0]
    k1 = int(math.ceil(0.5 * n0))
    k2 = int(math.ceil(0.5 * k1))
    P0, P1, P2 = _rup(n0, 512), _rup(k1, 512), _rup(k2, 512)
    D0 = x.shape[1]

    src = edge_index[0]
    dst = edge_index[1]
    E = src.shape[0]
    Ep = _rup(E, 2048)
    srcp = jnp.pad(src, (0, Ep - E))
    dstp = jnp.pad(dst, (0, Ep - E), constant_values=P0)

    xp = jnp.pad(x, ((0, P0 - n0), (0, 0)))
    p0b = jnp.broadcast_to(p0[:, None], (p0.shape[0], 128))
    p1b = jnp.broadcast_to(p1[:, None], (p1.shape[0], 128))
    b0r, b1r, b2r = b0[None, :], b1[None, :], b2[None, :]
    b3r, b4r = b3[None, :], b4[None, :]

    (cnt0b,) = _sc_edge_agg(srcp, dstp, [jnp.ones((P0, 128), F32)], P0)
    y0 = _xw_apply(xp, W0, cnt0b)
    agg0a, agg0b = _sc_edge_agg(srcp, dstp, [y0[:, :128], y0[:, 128:]], P0)
    agg0 = jnp.concatenate([agg0a, agg0b], axis=1)
    x1 = _relu_combine(agg0, y0, cnt0b, b0r)
    res0 = x1

    sb0, st0 = _scores(x1, p0b)
    rank0 = _ranks(sb0, st0, n0)[:, 0]
    perm0, vals0 = _sc_topk_select(rank0, sb0[:, 0], k1, P1, P0 - 1)
    x2g = _sc_gather_rows(x1, perm0)
    vals0b = jnp.broadcast_to(vals0[:, None], (P1, 128))

    inv0 = jnp.full((P0,), -1, jnp.int32).at[perm0].set(
        jnp.arange(P1, dtype=jnp.int32), mode="drop")
    inv0 = inv0.at[P0 - 1].set(-1)
    au = jnp.concatenate([src, perm0])
    av = jnp.concatenate([dst, perm0])
    Mr = _build_restricted(inv0, au, av, P1, P0)
    Mtr = _build_restricted(inv0, av, au, P1, P0)
    S1 = _mm_bt(Mr, Mtr, zero_diag=True, bm=1024, bn=1024, bk=512)
    S1T = _transpose(S1)
    cnt1b = _rowsum(S1T)

    y1 = _xw_apply(x2g, W1, cnt1b, scale=vals0b)
    x3 = _conv_apply(S1T, y1, cnt1b, b1r)
    res1 = x3

    sb1, st1 = _scores(x3, p1b)
    rank1 = _ranks(sb1, st1, k1)[:, 0]
    perm1, vals1 = _sc_topk_select(rank1, sb1[:, 0], k2, P2, P1 - 1)
    vals1b = jnp.broadcast_to(vals1[:, None], (P2, 128))

    G2, H2, x4g = _sc_gather_multi([S1, S1T, x3], perm1)
    perm1b = jnp.broadcast_to(perm1[:, None], (P2, 128))
    S2 = _mm_bt(G2, H2, zero_diag=True, diagp=perm1b,
                bm=512, bn=512, bk=512)
    S2T = _transpose(S2)
    cnt2b = _rowsum(S2T)

    y2 = _xw_apply(x4g, W2, cnt2b, scale=vals1b)
    x5 = _conv_apply(S2T, y2, cnt2b, b2r)

    h1 = _sc_up_add(res1, x5, perm1)
    y3 = _xw_apply(h1, W3, cnt1b)
    x6 = _conv_apply(S1T, y3, cnt1b, b3r)

    h0 = _sc_up_add(res0, x6, perm0)
    y4 = _xw_apply(h0, W4, cnt0b)
    (agg4,) = _sc_edge_agg(srcp, dstp, [y4], P0)
    x7 = _relu_combine(agg4, y4, cnt0b, b4r)
    return x7[:n0]


def _relu_kernel(agg_ref, y_ref, cnt_ref, b_ref, o_ref):
    dinv = lax.rsqrt(cnt_ref[:, 0:1] + 1.0)
    bias = jnp.broadcast_to(b_ref[...], o_ref.shape)
    o_ref[...] = jnp.maximum(dinv * (agg_ref[...] + y_ref[...]) + bias, 0.0)


def _relu_combine(agg, y, cnt_b, bias, bm=512):
    M, F = y.shape
    bn = F
    bm = min(bm, M)
    return pl.pallas_call(
        _relu_kernel,
        grid=(M // bm, F // bn),
        in_specs=[
            pl.BlockSpec((bm, bn), lambda i, j: (i, j)),
            pl.BlockSpec((bm, bn), lambda i, j: (i, j)),
            pl.BlockSpec((bm, 128), lambda i, j: (i, 0)),
            pl.BlockSpec((1, bn), lambda i, j: (0, j)),
        ],
        out_specs=pl.BlockSpec((bm, bn), lambda i, j: (i, j)),
        out_shape=jax.ShapeDtypeStruct((M, F), F32),
    )(agg, y, cnt_b, bias)


# speedup vs baseline: 1.0158x; 1.0158x over previous
import functools
import math

import jax
import jax.numpy as jnp
from jax import lax
from jax.experimental import pallas as pl
from jax.experimental.pallas import tpu as pltpu
from jax.experimental.pallas import tpu_sc as plsc

F32 = jnp.float32
_NC, _NS, _NW = 2, 16, 32


def _sc_mesh():
    return plsc.VectorSubcoreMesh(core_axis_name="c", subcore_axis_name="s",
                                  num_cores=_NC, num_subcores=_NS)


def _rup(x, m):
    return (x + m - 1) // m * m


def _mm_bt_kernel(a_ref, b_ref, o_ref, *, nk, zero_diag, bm, bn, bk,
                  pa_ref=None, pb_ref=None):
    k = pl.program_id(2)
    i, j = pl.program_id(0), pl.program_id(1)

    @pl.when(k == 0)
    def _():
        o_ref[...] = jnp.zeros_like(o_ref)

    a = a_ref[...]
    b = b_ref[...]
    if pa_ref is not None:
        kca = lax.broadcasted_iota(jnp.int32, (bm, bk), 1) + k * bk
        a = a + jnp.where(kca == pa_ref[:, 0:1], 1.0, 0.0)
        kcb = lax.broadcasted_iota(jnp.int32, (bn, bk), 1) + k * bk
        b = b + jnp.where(kcb == pb_ref[:, 0:1], 1.0, 0.0)
    if a.dtype != jnp.bfloat16:
        a = a.astype(jnp.bfloat16)
        b = b.astype(jnp.bfloat16)
    o_ref[...] += lax.dot_general(
        a, b, (((1,), (1,)), ((), ())), preferred_element_type=F32)

    if zero_diag:
        @pl.when(k == nk - 1)
        def _():
            ri = lax.broadcasted_iota(jnp.int32, (bm, bn), 0) + i * bm
            ci = lax.broadcasted_iota(jnp.int32, (bm, bn), 1) + j * bn
            o_ref[...] = jnp.where(ri == ci, 0.0, o_ref[...])


def _mm_bt_kernel_d(a_ref, b_ref, pa_ref, pb_ref, o_ref, **kw):
    _mm_bt_kernel(a_ref, b_ref, o_ref, pa_ref=pa_ref, pb_ref=pb_ref, **kw)


def _mm_bt(A, B, zero_diag=False, diagp=None, bm=256, bn=256, bk=1024):
    M, K = A.shape
    N, _ = B.shape
    bm, bn, bk = min(bm, M), min(bn, N), min(bk, K)
    nk = K // bk
    in_specs = [
        pl.BlockSpec((bm, bk), lambda i, j, k: (i, k)),
        pl.BlockSpec((bn, bk), lambda i, j, k: (j, k)),
    ]
    args = [A, B]
    if diagp is not None:
        in_specs += [
            pl.BlockSpec((bm, 128), lambda i, j, k: (i, 0)),
            pl.BlockSpec((bn, 128), lambda i, j, k: (j, 0)),
        ]
        args += [diagp, diagp]
        kfn = functools.partial(_mm_bt_kernel_d, nk=nk, zero_diag=zero_diag,
                                bm=bm, bn=bn, bk=bk)
    else:
        kfn = functools.partial(_mm_bt_kernel, nk=nk, zero_diag=zero_diag,
                                bm=bm, bn=bn, bk=bk)
    return pl.pallas_call(
        kfn,
        grid=(M // bm, N // bn, nk),
        in_specs=in_specs,
        out_specs=pl.BlockSpec((bm, bn), lambda i, j, k: (i, j)),
        out_shape=jax.ShapeDtypeStruct((M, N), F32),
    )(*args)


def _conv_kernel(st_ref, y_ref, ye_ref, cnt_ref, b_ref, o_ref, *, nk, bm, bn):
    k = pl.program_id(2)

    @pl.when(k == 0)
    def _():
        o_ref[...] = jnp.zeros_like(o_ref)

    o_ref[...] += lax.dot_general(
        st_ref[...], y_ref[...], (((1,), (0,)), ((), ())),
        preferred_element_type=F32)

    @pl.when(k == nk - 1)
    def _():
        dinv = lax.rsqrt(cnt_ref[:, 0:1] + 1.0)
        bias = jnp.broadcast_to(b_ref[...], (bm, bn))
        o_ref[...] = jnp.maximum(dinv * (o_ref[...] + ye_ref[...]) + bias, 0.0)


def _conv_apply(StT, y, cnt_b, bias, bm=512, bk=1024):
    M, K = StT.shape
    _, F = y.shape
    bn = F
    bm = min(bm, M)
    bk = min(bk, K)
    while K % bk:
        bk //= 2
    nk = K // bk
    return pl.pallas_call(
        functools.partial(_conv_kernel, nk=nk, bm=bm, bn=bn),
        grid=(M // bm, F // bn, nk),
        in_specs=[
            pl.BlockSpec((bm, bk), lambda i, j, k: (i, k)),
            pl.BlockSpec((bk, bn), lambda i, j, k: (k, j)),
            pl.BlockSpec((bm, bn), lambda i, j, k: (i, j)),
            pl.BlockSpec((bm, 128), lambda i, j, k: (i, 0)),
            pl.BlockSpec((1, bn), lambda i, j, k: (0, j)),
        ],
        out_specs=pl.BlockSpec((bm, bn), lambda i, j, k: (i, j)),
        out_shape=jax.ShapeDtypeStruct((M, F), F32),
    )(StT, y, y, cnt_b, bias)


def _xw_kernel(x_ref, w_ref, cnt_ref, o_ref):
    acc = lax.dot_general(x_ref[...], w_ref[...], (((1,), (0,)), ((), ())),
                          preferred_element_type=F32)
    dinv = lax.rsqrt(cnt_ref[:, 0:1] + 1.0)
    o_ref[...] = acc * dinv


def _xw_kernels(x_ref, sc_ref, w_ref, cnt_ref, o_ref):
    xin = x_ref[...] * sc_ref[:, 0:1]
    acc = lax.dot_general(xin, w_ref[...], (((1,), (0,)), ((), ())),
                          preferred_element_type=F32)
    dinv = lax.rsqrt(cnt_ref[:, 0:1] + 1.0)
    o_ref[...] = acc * dinv


def _xw_apply(x, W, cnt_b, scale=None, bm=512):
    M, K = x.shape
    _, F = W.shape
    bn = F
    bm = min(bm, M)
    args = [x] + ([scale] if scale is not None else []) + [W, cnt_b]
    in_specs = [pl.BlockSpec((bm, K), lambda i, j: (i, 0))]
    if scale is not None:
        in_specs.append(pl.BlockSpec((bm, 128), lambda i, j: (i, 0)))
    in_specs += [
        pl.BlockSpec((K, bn), lambda i, j: (0, j)),
        pl.BlockSpec((bm, 128), lambda i, j: (i, 0)),
    ]
    kfn = _xw_kernels if scale is not None else _xw_kernel
    return pl.pallas_call(
        kfn,
        grid=(M // bm, F // bn),
        in_specs=in_specs,
        out_specs=pl.BlockSpec((bm, bn), lambda i, j: (i, j)),
        out_shape=jax.ShapeDtypeStruct((M, F), F32),
    )(*args)


def _score_kernel(x_ref, p_ref, sb_ref, st_ref):
    pcol = p_ref[:, 0:1]
    ns = jnp.sum(pcol * pcol)
    xp = lax.dot_general(x_ref[...], p_ref[...], (((1,), (0,)), ((), ())),
                         preferred_element_type=F32)
    s = jnp.tanh(xp / jnp.sqrt(ns))
    sb_ref[...] = s
    st_ref[...] = s.T


def _scores(x, p_b, bm=512):
    M, K = x.shape
    bm = min(bm, M)
    return pl.pallas_call(
        _score_kernel,
        grid=(M // bm,),
        in_specs=[
            pl.BlockSpec((bm, K), lambda i: (i, 0)),
            pl.BlockSpec((K, 128), lambda i: (0, 0)),
        ],
        out_specs=[
            pl.BlockSpec((bm, 128), lambda i: (i, 0)),
            pl.BlockSpec((128, bm), lambda i: (0, i)),
        ],
        out_shape=[
            jax.ShapeDtypeStruct((M, 128), F32),
            jax.ShapeDtypeStruct((128, M), F32),
        ],
    )(x, p_b)


def _rank_kernel(sb_ref, st_ref, o_ref, *, bi, bj, nj, n_real):
    j = pl.program_id(1)

    @pl.when(j == 0)
    def _():
        o_ref[...] = jnp.zeros_like(o_ref)

    si = sb_ref[:, 0:1]
    sj = st_ref[0:1, :]
    ig = lax.broadcasted_iota(jnp.int32, (bi, bj), 0) + pl.program_id(0) * bi
    jg = lax.broadcasted_iota(jnp.int32, (bi, bj), 1) + j * bj
    valid = jg < n_real
    cmp = (sj > si) | ((sj == si) & (jg < ig))
    cnt = jnp.sum(jnp.where(cmp & valid, 1, 0), axis=1, keepdims=True)
    o_ref[...] += jnp.broadcast_to(cnt, o_ref.shape)

    @pl.when(j == nj - 1)
    def _():
        pad = jnp.where(ig[:, 0:1] >= n_real, jnp.int32(1 << 28), 0)
        o_ref[...] += jnp.broadcast_to(pad, o_ref.shape)


def _ranks(sb, st, n_real, bi=256, bj=512):
    M = sb.shape[0]
    bi, bj = min(bi, M), min(bj, M)
    nj = M // bj
    return pl.pallas_call(
        functools.partial(_rank_kernel, bi=bi, bj=bj, nj=nj, n_real=n_real),
        grid=(M // bi, nj),
        in_specs=[
            pl.BlockSpec((bi, 128), lambda i, j: (i, 0)),
            pl.BlockSpec((128, bj), lambda i, j: (0, j)),
        ],
        out_specs=pl.BlockSpec((bi, 128), lambda i, j: (i, 0)),
        out_shape=jax.ShapeDtypeStruct((M, 128), jnp.int32),
    )(sb, st)


def _tr_kernel(i_ref, o_ref):
    o_ref[...] = i_ref[...].T


def _transpose(A, bm=256, bn=256):
    M, N = A.shape
    bm, bn = min(bm, M), min(bn, N)
    return pl.pallas_call(
        _tr_kernel,
        grid=(M // bm, N // bn),
        in_specs=[pl.BlockSpec((bm, bn), lambda i, j: (i, j))],
        out_specs=pl.BlockSpec((bn, bm), lambda i, j: (j, i)),
        out_shape=jax.ShapeDtypeStruct((N, M), F32),
    )(A)


def _rowsum_kernel(a_ref, o_ref, *, nk):
    k = pl.program_id(1)

    @pl.when(k == 0)
    def _():
        o_ref[...] = jnp.zeros_like(o_ref)

    s = jnp.sum(a_ref[...], axis=1, keepdims=True)
    o_ref[...] += jnp.broadcast_to(s, o_ref.shape)


def _rowsum(A, bm=256, bk=2560):
    M, K = A.shape
    bm, bk = min(bm, M), min(bk, K)
    nk = K // bk
    return pl.pallas_call(
        functools.partial(_rowsum_kernel, nk=nk),
        grid=(M // bm, nk),
        in_specs=[pl.BlockSpec((bm, bk), lambda i, k: (i, k))],
        out_specs=pl.BlockSpec((bm, 128), lambda i, k: (i, 0)),
        out_shape=jax.ShapeDtypeStruct((M, 128), F32),
    )(A)


def _sc_gather_multi(tables, idx):
    B = idx.shape[0]
    nt = len(tables)
    bpw = B // _NW
    chunks = []
    for t in tables:
        D = t.shape[1]
        C = min(bpw, 128)
        while C * D * 4 > 196608 or bpw % C:
            C -= 1
        chunks.append(C)
    ipad = _rup(bpw + 16, 16)

    @functools.partial(
        pl.kernel, mesh=_sc_mesh(),
        out_type=[jax.ShapeDtypeStruct((B, t.shape[1]), t.dtype)
                  for t in tables],
        scratch_types=[pltpu.VMEM((ipad,), jnp.int32)] + [
            pltpu.VMEM((chunks[t], tables[t].shape[1]), tables[t].dtype)
            for t in range(nt)] + [pltpu.SemaphoreType.DMA])
    def k(*refs):
        tabs_h = refs[:nt]
        idx_h = refs[nt]
        outs_h = refs[nt + 1:2 * nt + 1]
        idx_v = refs[2 * nt + 1]
        rows_vs = refs[2 * nt + 2:3 * nt + 2]
        sem = refs[3 * nt + 2]
        wid = lax.axis_index("s") * _NC + lax.axis_index("c")
        base = wid * bpw
        pltpu.sync_copy(idx_h.at[pl.ds(base, bpw)], idx_v.at[pl.ds(0, bpw)])

        for t in range(nt):
            C = chunks[t]
            rows_v = rows_vs[t]

            def body(g, t=t, C=C, rows_v=rows_v):
                pltpu.async_copy(tabs_h[t].at[idx_v.at[pl.ds(g * C, C)]],
                                 rows_v, sem).wait()
                pltpu.sync_copy(rows_v, outs_h[t].at[pl.ds(base + g * C, C)])

            lax.fori_loop(0, bpw // C,
                          lambda g, _, b=body: (b(g), 0)[1], 0)

    return k(*tables, idx)


def _sc_gather_rows(table, idx):
    return _sc_gather_multi([table], idx)[0]


def _sc_topk_select(rank, s, k, kp, dump):
    M = rank.shape[0]

    @functools.partial(
        pl.kernel, mesh=_sc_mesh(),
        out_type=[jax.ShapeDtypeStruct((kp,), jnp.int32),
                  jax.ShapeDtypeStruct((kp,), F32)],
        scratch_types=[
            pltpu.VMEM((M,), jnp.int32),
            pltpu.VMEM((M,), F32),
            pltpu.VMEM((M,), jnp.int32),
            pltpu.VMEM((M // 128, 128), jnp.int32),
            pltpu.VMEM((kp + 128,), jnp.int32),
            pltpu.VMEM((kp + 128,), F32),
            pltpu.VMEM_SHARED((kp + 128,), jnp.int32),
            pltpu.VMEM_SHARED((kp + 128,), F32),
        ])
    def kk(rank_h, s_h, perm_h, vals_h, r_v, s_v, id_v, tgt_v, pi_v, vi_v,
           p_v, v_v):
        wid = lax.axis_index("s") * _NC + lax.axis_index("c")

        @pl.when(wid == 0)
        def _():
            pltpu.sync_copy(rank_h, r_v)
            pltpu.sync_copy(s_h, s_v)

            def init(i, _):
                pi_v[pl.ds(i * 16, 16)] = jnp.full((16,), dump, jnp.int32)
                vi_v[pl.ds(i * 16, 16)] = jnp.zeros((16,), F32)
                return 0

            lax.fori_loop(0, (kp + 128) // 16, init, 0)
            pltpu.sync_copy(pi_v, p_v)
            pltpu.sync_copy(vi_v, v_v)

            def prep(i, _):
                r = r_v[pl.ds(i * 16, 16)]
                tq = jnp.where(r < k, r, jnp.int32(kp))
                tgt_v[i // 8, pl.ds((i % 8) * 16, 16)] = tq
                id_v[pl.ds(i * 16, 16)] = lax.iota(jnp.int32, 16) + i * 16
                return 0

            lax.fori_loop(0, M // 16, prep, 0)

            def scat(j, _):
                pltpu.sync_copy(id_v.at[pl.ds(j * 128, 128)],
                                p_v.at[tgt_v.at[j]])
                pltpu.sync_copy(s_v.at[pl.ds(j * 128, 128)],
                                v_v.at[tgt_v.at[j]])
                return 0

            lax.fori_loop(0, M // 128, scat, 0)
            pltpu.sync_copy(p_v.at[pl.ds(0, kp)], perm_h)
            pltpu.sync_copy(v_v.at[pl.ds(0, kp)], vals_h)

    return kk(rank, s)


def _sc_up_add(res, xup, perm):
    M, F = res.shape
    B = perm.shape[0]
    bpw = B // _NS
    half = M // _NC
    rpt = M // _NW
    CC = 64
    while rpt % CC or CC * F * 4 > 131072:
        CC //= 2
    nsc = bpw // 80

    @functools.partial(
        pl.kernel, mesh=_sc_mesh(),
        out_type=jax.ShapeDtypeStruct((M + 8, F), F32),
        scratch_types=[
            pltpu.VMEM((_rup(bpw, 16),), jnp.int32),
            pltpu.VMEM((nsc, 80), jnp.int32),
            pltpu.VMEM((80, F), F32),
            pltpu.VMEM((80, F), F32),
            pltpu.SemaphoreType.DMA,
        ])
    def k(res_h, xup_h, perm_h, out_h, idx_v, tgt_v, xu_v, gat_v, sem):
        cid = lax.axis_index("c")
        sid = lax.axis_index("s")
        wid2 = cid * _NS + sid

        def cbody(t):
            st = wid2 * rpt + t * CC
            pltpu.sync_copy(res_h.at[pl.ds(st, CC)], gat_v.at[pl.ds(0, CC)])
            pltpu.sync_copy(gat_v.at[pl.ds(0, CC)], out_h.at[pl.ds(st, CC)])

        lax.fori_loop(0, rpt // CC, lambda t, _: (cbody(t), 0)[1], 0)
        plsc.subcore_barrier()

        base = sid * bpw
        pltpu.sync_copy(perm_h.at[pl.ds(base, bpw)],
                        idx_v.at[pl.ds(0, bpw)])
        lo = cid * half

        def abody(i):
            p = idx_v[pl.ds(i * 16, 16)]
            inh = (p >= lo) & (p < lo + half)
            tq = jnp.where(inh, p, jnp.int32(M))
            tgt_v[i // 5, pl.ds((i % 5) * 16, 16)] = tq

        lax.fori_loop(0, bpw // 16, lambda i, _: (abody(i), 0)[1], 0)

        nv = F // 16

        def sbody(r):
            pltpu.sync_copy(xup_h.at[pl.ds(base + r * 80, 80)], xu_v)
            pltpu.async_copy(res_h.at[idx_v.at[pl.ds(r * 80, 80)]],
                             gat_v, sem).wait()

            def inner(j, _):
                gat_v[j // nv, pl.ds((j % nv) * 16, 16)] = (
                    gat_v[j // nv, pl.ds((j % nv) * 16, 16)]
                    + xu_v[j // nv, pl.ds((j % nv) * 16, 16)])
                return 0

            lax.fori_loop(0, 80 * nv, inner, 0)
            pltpu.async_copy(gat_v, out_h.at[tgt_v.at[r]], sem).wait()

        lax.fori_loop(0, nsc, lambda r, _: (sbody(r), 0)[1], 0)

    return k(res, xup, perm)[:M]


def _sc_edge_agg(src, dst, ys, n_out):
    E = src.shape[0]
    F = 128
    nys = [y for y in ys if y is not None]
    nyt = len(nys)
    nph = len(ys)
    half = n_out // _NC
    ept = E // _NS
    nch = ept // 64
    arows = half + 128
    fpt = half // _NS

    @functools.partial(
        pl.kernel, mesh=_sc_mesh(),
        out_type=[jax.ShapeDtypeStruct((n_out, F), F32) for _ in ys],
        scratch_types=[
            pltpu.VMEM((ept,), jnp.int32),
            pltpu.VMEM((ept,), jnp.int32),
            pltpu.VMEM((nch, 64), jnp.int32),
            pltpu.VMEM((64, F), F32),
            pltpu.VMEM((64, F), F32),
            pltpu.VMEM((64, F), F32),
            pltpu.SemaphoreType.DMA,
            pltpu.SemaphoreType.DMA,
            pltpu.VMEM_SHARED((arows, F), F32),
        ])
    def k(src_h, dst_h, *rest):
        ys_in = rest[:nyt]
        ys_h = []
        q = 0
        for y in ys:
            if y is None:
                ys_h.append(None)
            else:
                ys_h.append(ys_in[q])
                q += 1
        outs_h = rest[nyt:nyt + nph]
        src_v, dst_v, tgt_v, zb_v, ra_v, rb_v, sema, semb, acc = (
            rest[nyt + nph:])
        cid = lax.axis_index("c")
        sid = lax.axis_index("s")
        lo = cid * half

        def zrow(j, _):
            zb_v[j // 8, pl.ds((j % 8) * 16, 16)] = jnp.zeros((16,), F32)
            return 0

        lax.fori_loop(0, 64 * 8, zrow, 0)

        eb = sid * ept
        pltpu.sync_copy(src_h.at[pl.ds(eb, ept)], src_v)
        pltpu.sync_copy(dst_h.at[pl.ds(eb, ept)], dst_v)

        def tb(i):
            d = dst_v[pl.ds(i * 16, 16)]
            inh = (d >= lo) & (d < lo + half)
            tq = jnp.where(inh, d - lo, jnp.int32(half))
            tgt_v[i // 4, pl.ds((i % 4) * 16, 16)] = tq

        lax.fori_loop(0, ept // 16, lambda i, _: (tb(i), 0)[1], 0)

        zpt = arows // _NS

        for f in range(nph):
            y_h = ys_h[f]
            out_h = outs_h[f]

            def zc2(u, _):
                pltpu.sync_copy(zb_v, acc.at[pl.ds(sid * zpt + u * 64, 64)])
                return 0

            lax.fori_loop(0, zpt // 64, zc2, 0)
            pltpu.sync_copy(
                zb_v.at[pl.ds(0, zpt - (zpt // 64) * 64)],
                acc.at[pl.ds(sid * zpt + (zpt // 64) * 64,
                             zpt - (zpt // 64) * 64)])
            plsc.subcore_barrier()

            if y_h is None:
                def orow(j, _):
                    ra_v[j // 8, pl.ds((j % 8) * 16, 16)] = (
                        jnp.ones((16,), F32))
                    return 0

                lax.fori_loop(0, 64 * 8, orow, 0)

                def obody(g, _):
                    pltpu.sync_copy(ra_v, acc.at[tgt_v.at[g]], add=True)
                    return 0

                lax.fori_loop(0, nch, obody, 0)
            else:
                def gath(g, buf, sem):
                    return pltpu.async_copy(
                        y_h.at[src_v.at[pl.ds(g * 64, 64)]], buf, sem)

                gath(0, ra_v, sema).wait()

                def pbody(g0):
                    hb = gath(g0 + 1, rb_v, semb)
                    pltpu.sync_copy(ra_v, acc.at[tgt_v.at[g0]], add=True)
                    g2 = jnp.minimum(g0 + 2, nch - 2)
                    ha = gath(g2, ra_v, sema)
                    hb.wait()
                    pltpu.sync_copy(rb_v, acc.at[tgt_v.at[g0 + 1]], add=True)
                    ha.wait()

                lax.fori_loop(0, nch // 2,
                              lambda t, _: (pbody(t * 2), 0)[1], 0)
            plsc.subcore_barrier()

            def fcp(u, _):
                st = sid * fpt + u * 64
                pltpu.sync_copy(acc.at[pl.ds(st, 64)],
                                out_h.at[pl.ds(lo + st, 64)])
                return 0

            lax.fori_loop(0, fpt // 64, fcp, 0)
            plsc.subcore_barrier()

    return k(src, dst, *nys)


def _scatter_counts(dst, n):
    return jnp.zeros((n,), F32).at[dst].add(1.0, mode="drop")


def _topk_select(rank, s, k, kp, dump):
    sel = rank < k
    idx = jnp.arange(rank.shape[0], dtype=jnp.int32)
    tgt = jnp.where(sel, rank, k)
    perm = jnp.full((kp + 1,), dump, jnp.int32).at[tgt].set(
        jnp.where(sel, idx, dump), mode="drop")
    vals = jnp.zeros((kp + 1,), F32).at[tgt].set(
        jnp.where(sel, s, 0.0), mode="drop")
    perm = perm.at[k].set(dump) if k < kp else perm
    vals = vals.at[k].set(0.0) if k < kp else vals
    return perm[:kp], vals[:kp]


def _build_restricted(inv, rows_idx, cols_idx, kp, n):
    a = inv[rows_idx]
    w = jnp.where(a >= 0, 1.0, 0.0).astype(jnp.bfloat16)
    a = jnp.where(a >= 0, a, kp)
    out = jnp.zeros((kp + 1, n), jnp.bfloat16).at[a, cols_idx].add(
        w, mode="drop")
    return out[:kp]


def kernel(x, edge_index, W0, b0, W1, b1, W2, b2, W3, b3, W4, b4, p0, p1):
    n0 = x.shape[0]
    k1 = int(math.ceil(0.5 * n0))
    k2 = int(math.ceil(0.5 * k1))
    P0, P1, P2 = _rup(n0, 512), _rup(k1, 512), _rup(k2, 512)
    D0 = x.shape[1]

    src = edge_index[0]
    dst = edge_index[1]
    E = src.shape[0]
    Ep = _rup(E, 2048)
    srcp = jnp.pad(src, (0, Ep - E))
    dstp = jnp.pad(dst, (0, Ep - E), constant_values=P0)

    xp = jnp.pad(x, ((0, P0 - n0), (0, 0)))
    p0b = jnp.broadcast_to(p0[:, None], (p0.shape[0], 128))
    p1b = jnp.broadcast_to(p1[:, None], (p1.shape[0], 128))
    b0r, b1r, b2r = b0[None, :], b1[None, :], b2[None, :]
    b3r, b4r = b3[None, :], b4[None, :]

    (cnt0b,) = _sc_edge_agg(srcp, dstp, [None], P0)
    y0 = _xw_apply(xp, W0, cnt0b)
    agg0a, agg0b = _sc_edge_agg(srcp, dstp, [y0[:, :128], y0[:, 128:]], P0)
    agg0 = jnp.concatenate([agg0a, agg0b], axis=1)
    x1 = _relu_combine(agg0, y0, cnt0b, b0r)
    res0 = x1

    sb0, st0 = _scores(x1, p0b)
    rank0 = _ranks(sb0, st0, n0)[:, 0]
    perm0, vals0 = _sc_topk_select(rank0, sb0[:, 0], k1, P1, P0 - 1)
    x2g = _sc_gather_rows(x1, perm0)
    vals0b = jnp.broadcast_to(vals0[:, None], (P1, 128))

    inv0 = jnp.full((P0,), -1, jnp.int32).at[perm0].set(
        jnp.arange(P1, dtype=jnp.int32), mode="drop")
    inv0 = inv0.at[P0 - 1].set(-1)
    au = jnp.concatenate([src, perm0])
    av = jnp.concatenate([dst, perm0])
    Mr = _build_restricted(inv0, au, av, P1, P0)
    Mtr = _build_restricted(inv0, av, au, P1, P0)
    S1 = _mm_bt(Mr, Mtr, zero_diag=True, bm=1024, bn=1024, bk=512)
    S1T = _transpose(S1)
    cnt1b = _rowsum(S1T)

    y1 = _xw_apply(x2g, W1, cnt1b, scale=vals0b)
    x3 = _conv_apply(S1T, y1, cnt1b, b1r)
    res1 = x3

    sb1, st1 = _scores(x3, p1b)
    rank1 = _ranks(sb1, st1, k1)[:, 0]
    perm1, vals1 = _sc_topk_select(rank1, sb1[:, 0], k2, P2, P1 - 1)
    vals1b = jnp.broadcast_to(vals1[:, None], (P2, 128))

    G2, H2, x4g = _sc_gather_multi([S1, S1T, x3], perm1)
    perm1b = jnp.broadcast_to(perm1[:, None], (P2, 128))
    S2 = _mm_bt(G2, H2, zero_diag=True, diagp=perm1b,
                bm=512, bn=512, bk=512)
    S2T = _transpose(S2)
    cnt2b = _rowsum(S2T)

    y2 = _xw_apply(x4g, W2, cnt2b, scale=vals1b)
    x5 = _conv_apply(S2T, y2, cnt2b, b2r)

    h1 = _sc_up_add(res1, x5, perm1)
    y3 = _xw_apply(h1, W3, cnt1b)
    x6 = _conv_apply(S1T, y3, cnt1b, b3r)

    h0 = _sc_up_add(res0, x6, perm0)
    y4 = _xw_apply(h0, W4, cnt0b)
    (agg4,) = _sc_edge_agg(srcp, dstp, [y4], P0)
    x7 = _relu_combine(agg4, y4, cnt0b, b4r)
    return x7[:n0]


def _relu_kernel(agg_ref, y_ref, cnt_ref, b_ref, o_ref):
    dinv = lax.rsqrt(cnt_ref[:, 0:1] + 1.0)
    bias = jnp.broadcast_to(b_ref[...], o_ref.shape)
    o_ref[...] = jnp.maximum(dinv * (agg_ref[...] + y_ref[...]) + bias, 0.0)


def _relu_combine(agg, y, cnt_b, bias, bm=512):
    M, F = y.shape
    bn = F
    bm = min(bm, M)
    return pl.pallas_call(
        _relu_kernel,
        grid=(M // bm, F // bn),
        in_specs=[
            pl.BlockSpec((bm, bn), lambda i, j: (i, j)),
            pl.BlockSpec((bm, bn), lambda i, j: (i, j)),
            pl.BlockSpec((bm, 128), lambda i, j: (i, 0)),
            pl.BlockSpec((1, bn), lambda i, j: (0, j)),
        ],
        out_specs=pl.BlockSpec((bm, bn), lambda i, j: (i, j)),
        out_shape=jax.ShapeDtypeStruct((M, F), F32),
    )(agg, y, cnt_b, bias)


# speedup vs baseline: 1.0184x; 1.0026x over previous
import functools
import math

import jax
import jax.numpy as jnp
from jax import lax
from jax.experimental import pallas as pl
from jax.experimental.pallas import tpu as pltpu
from jax.experimental.pallas import tpu_sc as plsc

F32 = jnp.float32
_NC, _NS, _NW = 2, 16, 32


def _sc_mesh():
    return plsc.VectorSubcoreMesh(core_axis_name="c", subcore_axis_name="s",
                                  num_cores=_NC, num_subcores=_NS)


def _rup(x, m):
    return (x + m - 1) // m * m


def _mm_bt_kernel(a_ref, b_ref, o_ref, *, nk, zero_diag, bm, bn, bk,
                  pa_ref=None, pb_ref=None):
    k = pl.program_id(2)
    i, j = pl.program_id(0), pl.program_id(1)

    @pl.when(k == 0)
    def _():
        o_ref[...] = jnp.zeros_like(o_ref)

    a = a_ref[...]
    b = b_ref[...]
    if pa_ref is not None:
        kca = lax.broadcasted_iota(jnp.int32, (bm, bk), 1) + k * bk
        a = a + jnp.where(kca == pa_ref[:, 0:1], 1.0, 0.0)
        kcb = lax.broadcasted_iota(jnp.int32, (bn, bk), 1) + k * bk
        b = b + jnp.where(kcb == pb_ref[:, 0:1], 1.0, 0.0)
    if a.dtype != jnp.bfloat16:
        a = a.astype(jnp.bfloat16)
        b = b.astype(jnp.bfloat16)
    o_ref[...] += lax.dot_general(
        a, b, (((1,), (1,)), ((), ())), preferred_element_type=F32)

    if zero_diag:
        @pl.when(k == nk - 1)
        def _():
            ri = lax.broadcasted_iota(jnp.int32, (bm, bn), 0) + i * bm
            ci = lax.broadcasted_iota(jnp.int32, (bm, bn), 1) + j * bn
            o_ref[...] = jnp.where(ri == ci, 0.0, o_ref[...])


def _mm_bt_kernel_d(a_ref, b_ref, pa_ref, pb_ref, o_ref, **kw):
    _mm_bt_kernel(a_ref, b_ref, o_ref, pa_ref=pa_ref, pb_ref=pb_ref, **kw)


def _mm_bt(A, B, zero_diag=False, diagp=None, bm=256, bn=256, bk=1024):
    M, K = A.shape
    N, _ = B.shape
    bm, bn, bk = min(bm, M), min(bn, N), min(bk, K)
    nk = K // bk
    in_specs = [
        pl.BlockSpec((bm, bk), lambda i, j, k: (i, k)),
        pl.BlockSpec((bn, bk), lambda i, j, k: (j, k)),
    ]
    args = [A, B]
    if diagp is not None:
        in_specs += [
            pl.BlockSpec((bm, 128), lambda i, j, k: (i, 0)),
            pl.BlockSpec((bn, 128), lambda i, j, k: (j, 0)),
        ]
        args += [diagp, diagp]
        kfn = functools.partial(_mm_bt_kernel_d, nk=nk, zero_diag=zero_diag,
                                bm=bm, bn=bn, bk=bk)
    else:
        kfn = functools.partial(_mm_bt_kernel, nk=nk, zero_diag=zero_diag,
                                bm=bm, bn=bn, bk=bk)
    return pl.pallas_call(
        kfn,
        grid=(M // bm, N // bn, nk),
        in_specs=in_specs,
        out_specs=pl.BlockSpec((bm, bn), lambda i, j, k: (i, j)),
        out_shape=jax.ShapeDtypeStruct((M, N), F32),
    )(*args)


def _conv_kernel(st_ref, y_ref, ye_ref, cnt_ref, b_ref, o_ref, *, nk, bm, bn):
    k = pl.program_id(2)

    @pl.when(k == 0)
    def _():
        o_ref[...] = jnp.zeros_like(o_ref)

    o_ref[...] += lax.dot_general(
        st_ref[...], y_ref[...], (((1,), (0,)), ((), ())),
        preferred_element_type=F32)

    @pl.when(k == nk - 1)
    def _():
        dinv = lax.rsqrt(cnt_ref[:, 0:1] + 1.0)
        bias = jnp.broadcast_to(b_ref[...], (bm, bn))
        o_ref[...] = jnp.maximum(dinv * (o_ref[...] + ye_ref[...]) + bias, 0.0)


def _conv_apply(StT, y, cnt_b, bias, bm=512, bk=1024):
    M, K = StT.shape
    _, F = y.shape
    bn = F
    bm = min(bm, M)
    bk = min(bk, K)
    while K % bk:
        bk //= 2
    nk = K // bk
    return pl.pallas_call(
        functools.partial(_conv_kernel, nk=nk, bm=bm, bn=bn),
        grid=(M // bm, F // bn, nk),
        in_specs=[
            pl.BlockSpec((bm, bk), lambda i, j, k: (i, k)),
            pl.BlockSpec((bk, bn), lambda i, j, k: (k, j)),
            pl.BlockSpec((bm, bn), lambda i, j, k: (i, j)),
            pl.BlockSpec((bm, 128), lambda i, j, k: (i, 0)),
            pl.BlockSpec((1, bn), lambda i, j, k: (0, j)),
        ],
        out_specs=pl.BlockSpec((bm, bn), lambda i, j, k: (i, j)),
        out_shape=jax.ShapeDtypeStruct((M, F), F32),
    )(StT, y, y, cnt_b, bias)


def _xw_kernel(x_ref, w_ref, cnt_ref, o_ref):
    acc = lax.dot_general(x_ref[...], w_ref[...], (((1,), (0,)), ((), ())),
                          preferred_element_type=F32)
    dinv = lax.rsqrt(cnt_ref[:, 0:1] + 1.0)
    o_ref[...] = acc * dinv


def _xw_kernels(x_ref, sc_ref, w_ref, cnt_ref, o_ref):
    xin = x_ref[...] * sc_ref[:, 0:1]
    acc = lax.dot_general(xin, w_ref[...], (((1,), (0,)), ((), ())),
                          preferred_element_type=F32)
    dinv = lax.rsqrt(cnt_ref[:, 0:1] + 1.0)
    o_ref[...] = acc * dinv


def _xw_apply(x, W, cnt_b, scale=None, bm=512):
    M, K = x.shape
    _, F = W.shape
    bn = F
    bm = min(bm, M)
    args = [x] + ([scale] if scale is not None else []) + [W, cnt_b]
    in_specs = [pl.BlockSpec((bm, K), lambda i, j: (i, 0))]
    if scale is not None:
        in_specs.append(pl.BlockSpec((bm, 128), lambda i, j: (i, 0)))
    in_specs += [
        pl.BlockSpec((K, bn), lambda i, j: (0, j)),
        pl.BlockSpec((bm, 128), lambda i, j: (i, 0)),
    ]
    kfn = _xw_kernels if scale is not None else _xw_kernel
    return pl.pallas_call(
        kfn,
        grid=(M // bm, F // bn),
        in_specs=in_specs,
        out_specs=pl.BlockSpec((bm, bn), lambda i, j: (i, j)),
        out_shape=jax.ShapeDtypeStruct((M, F), F32),
    )(*args)


def _score_kernel(x_ref, p_ref, sb_ref, st_ref, *, bm, n_real):
    pcol = p_ref[:, 0:1]
    ns = jnp.sum(pcol * pcol)
    xp = lax.dot_general(x_ref[...], p_ref[...], (((1,), (0,)), ((), ())),
                         preferred_element_type=F32)
    s = jnp.tanh(xp / jnp.sqrt(ns))
    rg = (lax.broadcasted_iota(jnp.int32, (bm, 1), 0)
          + pl.program_id(0) * bm)
    s = jnp.where(rg < n_real, s, -2.0)
    sb_ref[...] = s
    st_ref[...] = s.T


def _scores(x, p_b, n_real, bm=512):
    M, K = x.shape
    bm = min(bm, M)
    return pl.pallas_call(
        functools.partial(_score_kernel, bm=bm, n_real=n_real),
        grid=(M // bm,),
        in_specs=[
            pl.BlockSpec((bm, K), lambda i: (i, 0)),
            pl.BlockSpec((K, 128), lambda i: (0, 0)),
        ],
        out_specs=[
            pl.BlockSpec((bm, 128), lambda i: (i, 0)),
            pl.BlockSpec((128, bm), lambda i: (0, i)),
        ],
        out_shape=[
            jax.ShapeDtypeStruct((M, 128), F32),
            jax.ShapeDtypeStruct((128, M), F32),
        ],
    )(x, p_b)


def _rank_kernel(sb_ref, st_ref, o_ref, *, bi, bj, nj, n_real):
    j = pl.program_id(1)

    @pl.when(j == 0)
    def _():
        o_ref[...] = jnp.zeros_like(o_ref)

    i = pl.program_id(0)
    si = sb_ref[:, 0:1]
    sj = st_ref[0:1, :]
    ilo = i * bi
    jlo = j * bj
    diag = (jlo < ilo + bi) & (ilo < jlo + bj)

    @pl.when(diag)
    def _():
        gt = sj > si
        ig = lax.broadcasted_iota(jnp.int32, (bi, bj), 0) + ilo
        jg = lax.broadcasted_iota(jnp.int32, (bi, bj), 1) + jlo
        cmp = gt | ((sj == si) & (jg < ig))
        cnt = jnp.sum(jnp.where(cmp, 1, 0), axis=1, keepdims=True)
        o_ref[...] += jnp.broadcast_to(cnt, o_ref.shape)

    @pl.when(jnp.logical_not(diag))
    def _():
        cgt = jnp.sum(jnp.where(sj > si, 1, 0), axis=1, keepdims=True)
        ceq = jnp.sum(jnp.where(sj == si, 1, 0), axis=1, keepdims=True)
        cnt = cgt + jnp.where(jlo < ilo, ceq, 0)
        o_ref[...] += jnp.broadcast_to(cnt, o_ref.shape)

    @pl.when(j == nj - 1)
    def _():
        pad = jnp.where(si <= -2.0, jnp.int32(1 << 28), 0)
        o_ref[...] += jnp.broadcast_to(pad, o_ref.shape)


def _ranks(sb, st, n_real, bi=256, bj=512):
    M = sb.shape[0]
    bi, bj = min(bi, M), min(bj, M)
    nj = M // bj
    return pl.pallas_call(
        functools.partial(_rank_kernel, bi=bi, bj=bj, nj=nj, n_real=n_real),
        grid=(M // bi, nj),
        in_specs=[
            pl.BlockSpec((bi, 128), lambda i, j: (i, 0)),
            pl.BlockSpec((128, bj), lambda i, j: (0, j)),
        ],
        out_specs=pl.BlockSpec((bi, 128), lambda i, j: (i, 0)),
        out_shape=jax.ShapeDtypeStruct((M, 128), jnp.int32),
    )(sb, st)


def _tr_kernel(i_ref, o_ref):
    o_ref[...] = i_ref[...].T


def _transpose(A, bm=256, bn=256):
    M, N = A.shape
    bm, bn = min(bm, M), min(bn, N)
    return pl.pallas_call(
        _tr_kernel,
        grid=(M // bm, N // bn),
        in_specs=[pl.BlockSpec((bm, bn), lambda i, j: (i, j))],
        out_specs=pl.BlockSpec((bn, bm), lambda i, j: (j, i)),
        out_shape=jax.ShapeDtypeStruct((N, M), F32),
    )(A)


def _rowsum_kernel(a_ref, o_ref, *, nk):
    k = pl.program_id(1)

    @pl.when(k == 0)
    def _():
        o_ref[...] = jnp.zeros_like(o_ref)

    s = jnp.sum(a_ref[...], axis=1, keepdims=True)
    o_ref[...] += jnp.broadcast_to(s, o_ref.shape)


def _rowsum(A, bm=256, bk=2560):
    M, K = A.shape
    bm, bk = min(bm, M), min(bk, K)
    nk = K // bk
    return pl.pallas_call(
        functools.partial(_rowsum_kernel, nk=nk),
        grid=(M // bm, nk),
        in_specs=[pl.BlockSpec((bm, bk), lambda i, k: (i, k))],
        out_specs=pl.BlockSpec((bm, 128), lambda i, k: (i, 0)),
        out_shape=jax.ShapeDtypeStruct((M, 128), F32),
    )(A)


def _sc_gather_multi(tables, idx):
    B = idx.shape[0]
    nt = len(tables)
    bpw = B // _NW
    chunks = []
    for t in tables:
        D = t.shape[1]
        C = min(bpw, 128)
        while C * D * 4 > 196608 or bpw % C:
            C -= 1
        chunks.append(C)
    ipad = _rup(bpw + 16, 16)

    @functools.partial(
        pl.kernel, mesh=_sc_mesh(),
        out_type=[jax.ShapeDtypeStruct((B, t.shape[1]), t.dtype)
                  for t in tables],
        scratch_types=[pltpu.VMEM((ipad,), jnp.int32)] + [
            pltpu.VMEM((chunks[t], tables[t].shape[1]), tables[t].dtype)
            for t in range(nt)] + [pltpu.SemaphoreType.DMA])
    def k(*refs):
        tabs_h = refs[:nt]
        idx_h = refs[nt]
        outs_h = refs[nt + 1:2 * nt + 1]
        idx_v = refs[2 * nt + 1]
        rows_vs = refs[2 * nt + 2:3 * nt + 2]
        sem = refs[3 * nt + 2]
        wid = lax.axis_index("s") * _NC + lax.axis_index("c")
        base = wid * bpw
        pltpu.sync_copy(idx_h.at[pl.ds(base, bpw)], idx_v.at[pl.ds(0, bpw)])

        for t in range(nt):
            C = chunks[t]
            rows_v = rows_vs[t]

            def body(g, t=t, C=C, rows_v=rows_v):
                pltpu.async_copy(tabs_h[t].at[idx_v.at[pl.ds(g * C, C)]],
                                 rows_v, sem).wait()
                pltpu.sync_copy(rows_v, outs_h[t].at[pl.ds(base + g * C, C)])

            lax.fori_loop(0, bpw // C,
                          lambda g, _, b=body: (b(g), 0)[1], 0)

    return k(*tables, idx)


def _sc_gather_rows(table, idx):
    return _sc_gather_multi([table], idx)[0]


def _sc_topk_select(rank, s, k, kp, dump):
    M = rank.shape[0]

    @functools.partial(
        pl.kernel, mesh=_sc_mesh(),
        out_type=[jax.ShapeDtypeStruct((kp,), jnp.int32),
                  jax.ShapeDtypeStruct((kp,), F32)],
        scratch_types=[
            pltpu.VMEM((M,), jnp.int32),
            pltpu.VMEM((M,), F32),
            pltpu.VMEM((M,), jnp.int32),
            pltpu.VMEM((M // 128, 128), jnp.int32),
            pltpu.VMEM((kp + 128,), jnp.int32),
            pltpu.VMEM((kp + 128,), F32),
            pltpu.VMEM_SHARED((kp + 128,), jnp.int32),
            pltpu.VMEM_SHARED((kp + 128,), F32),
        ])
    def kk(rank_h, s_h, perm_h, vals_h, r_v, s_v, id_v, tgt_v, pi_v, vi_v,
           p_v, v_v):
        wid = lax.axis_index("s") * _NC + lax.axis_index("c")

        @pl.when(wid == 0)
        def _():
            pltpu.sync_copy(rank_h, r_v)
            pltpu.sync_copy(s_h, s_v)

            def init(i, _):
                pi_v[pl.ds(i * 16, 16)] = jnp.full((16,), dump, jnp.int32)
                vi_v[pl.ds(i * 16, 16)] = jnp.zeros((16,), F32)
                return 0

            lax.fori_loop(0, (kp + 128) // 16, init, 0)
            pltpu.sync_copy(pi_v, p_v)
            pltpu.sync_copy(vi_v, v_v)

            def prep(i, _):
                r = r_v[pl.ds(i * 16, 16)]
                tq = jnp.where(r < k, r, jnp.int32(kp))
                tgt_v[i // 8, pl.ds((i % 8) * 16, 16)] = tq
                id_v[pl.ds(i * 16, 16)] = lax.iota(jnp.int32, 16) + i * 16
                return 0

            lax.fori_loop(0, M // 16, prep, 0)

            def scat(j, _):
                pltpu.sync_copy(id_v.at[pl.ds(j * 128, 128)],
                                p_v.at[tgt_v.at[j]])
                pltpu.sync_copy(s_v.at[pl.ds(j * 128, 128)],
                                v_v.at[tgt_v.at[j]])
                return 0

            lax.fori_loop(0, M // 128, scat, 0)
            pltpu.sync_copy(p_v.at[pl.ds(0, kp)], perm_h)
            pltpu.sync_copy(v_v.at[pl.ds(0, kp)], vals_h)

    return kk(rank, s)


def _sc_up_add(res, xup, perm):
    M, F = res.shape
    B = perm.shape[0]
    bpw = B // _NS
    half = M // _NC
    rpt = M // _NW
    CC = 64
    while rpt % CC or CC * F * 4 > 131072:
        CC //= 2
    nsc = bpw // 80

    @functools.partial(
        pl.kernel, mesh=_sc_mesh(),
        out_type=jax.ShapeDtypeStruct((M + 8, F), F32),
        scratch_types=[
            pltpu.VMEM((_rup(bpw, 16),), jnp.int32),
            pltpu.VMEM((nsc, 80), jnp.int32),
            pltpu.VMEM((80, F), F32),
            pltpu.VMEM((80, F), F32),
            pltpu.SemaphoreType.DMA,
        ])
    def k(res_h, xup_h, perm_h, out_h, idx_v, tgt_v, xu_v, gat_v, sem):
        cid = lax.axis_index("c")
        sid = lax.axis_index("s")
        wid2 = cid * _NS + sid

        def cbody(t):
            st = wid2 * rpt + t * CC
            pltpu.sync_copy(res_h.at[pl.ds(st, CC)], gat_v.at[pl.ds(0, CC)])
            pltpu.sync_copy(gat_v.at[pl.ds(0, CC)], out_h.at[pl.ds(st, CC)])

        lax.fori_loop(0, rpt // CC, lambda t, _: (cbody(t), 0)[1], 0)
        plsc.subcore_barrier()

        base = sid * bpw
        pltpu.sync_copy(perm_h.at[pl.ds(base, bpw)],
                        idx_v.at[pl.ds(0, bpw)])
        lo = cid * half

        def abody(i):
            p = idx_v[pl.ds(i * 16, 16)]
            inh = (p >= lo) & (p < lo + half)
            tq = jnp.where(inh, p, jnp.int32(M))
            tgt_v[i // 5, pl.ds((i % 5) * 16, 16)] = tq

        lax.fori_loop(0, bpw // 16, lambda i, _: (abody(i), 0)[1], 0)

        nv = F // 16

        def sbody(r):
            pltpu.sync_copy(xup_h.at[pl.ds(base + r * 80, 80)], xu_v)
            pltpu.async_copy(res_h.at[idx_v.at[pl.ds(r * 80, 80)]],
                             gat_v, sem).wait()

            def inner(j, _):
                gat_v[j // nv, pl.ds((j % nv) * 16, 16)] = (
                    gat_v[j // nv, pl.ds((j % nv) * 16, 16)]
                    + xu_v[j // nv, pl.ds((j % nv) * 16, 16)])
                return 0

            lax.fori_loop(0, 80 * nv, inner, 0)
            pltpu.async_copy(gat_v, out_h.at[tgt_v.at[r]], sem).wait()

        lax.fori_loop(0, nsc, lambda r, _: (sbody(r), 0)[1], 0)

    return k(res, xup, perm)[:M]


def _sc_edge_agg(src, dst, ys, n_out):
    E = src.shape[0]
    F = 128
    nys = [y for y in ys if y is not None]
    nyt = len(nys)
    nph = len(ys)
    half = n_out // _NC
    ept = E // _NS
    nch = ept // 64
    arows = half + 128
    fpt = half // _NS

    @functools.partial(
        pl.kernel, mesh=_sc_mesh(),
        out_type=[jax.ShapeDtypeStruct((n_out, F), F32) for _ in ys],
        scratch_types=[
            pltpu.VMEM((ept,), jnp.int32),
            pltpu.VMEM((ept,), jnp.int32),
            pltpu.VMEM((nch, 64), jnp.int32),
            pltpu.VMEM((64, F), F32),
            pltpu.VMEM((64, F), F32),
            pltpu.VMEM((64, F), F32),
            pltpu.SemaphoreType.DMA,
            pltpu.SemaphoreType.DMA,
            pltpu.VMEM_SHARED((arows, F), F32),
        ])
    def k(src_h, dst_h, *rest):
        ys_in = rest[:nyt]
        ys_h = []
        q = 0
        for y in ys:
            if y is None:
                ys_h.append(None)
            else:
                ys_h.append(ys_in[q])
                q += 1
        outs_h = rest[nyt:nyt + nph]
        src_v, dst_v, tgt_v, zb_v, ra_v, rb_v, sema, semb, acc = (
            rest[nyt + nph:])
        cid = lax.axis_index("c")
        sid = lax.axis_index("s")
        lo = cid * half

        def zrow(j, _):
            zb_v[j // 8, pl.ds((j % 8) * 16, 16)] = jnp.zeros((16,), F32)
            return 0

        lax.fori_loop(0, 64 * 8, zrow, 0)

        eb = sid * ept
        pltpu.sync_copy(src_h.at[pl.ds(eb, ept)], src_v)
        pltpu.sync_copy(dst_h.at[pl.ds(eb, ept)], dst_v)

        def tb(i):
            d = dst_v[pl.ds(i * 16, 16)]
            inh = (d >= lo) & (d < lo + half)
            tq = jnp.where(inh, d - lo, jnp.int32(half))
            tgt_v[i // 4, pl.ds((i % 4) * 16, 16)] = tq

        lax.fori_loop(0, ept // 16, lambda i, _: (tb(i), 0)[1], 0)

        zpt = arows // _NS

        for f in range(nph):
            y_h = ys_h[f]
            out_h = outs_h[f]

            def zc2(u, _):
                pltpu.sync_copy(zb_v, acc.at[pl.ds(sid * zpt + u * 64, 64)])
                return 0

            lax.fori_loop(0, zpt // 64, zc2, 0)
            pltpu.sync_copy(
                zb_v.at[pl.ds(0, zpt - (zpt // 64) * 64)],
                acc.at[pl.ds(sid * zpt + (zpt // 64) * 64,
                             zpt - (zpt // 64) * 64)])
            plsc.subcore_barrier()

            if y_h is None:
                def orow(j, _):
                    ra_v[j // 8, pl.ds((j % 8) * 16, 16)] = (
                        jnp.ones((16,), F32))
                    return 0

                lax.fori_loop(0, 64 * 8, orow, 0)

                def obody(g, _):
                    pltpu.sync_copy(ra_v, acc.at[tgt_v.at[g]], add=True)
                    return 0

                lax.fori_loop(0, nch, obody, 0)
            else:
                def gath(g, buf, sem):
                    return pltpu.async_copy(
                        y_h.at[src_v.at[pl.ds(g * 64, 64)]], buf, sem)

                gath(0, ra_v, sema).wait()

                def pbody(g0):
                    hb = gath(g0 + 1, rb_v, semb)
                    pltpu.sync_copy(ra_v, acc.at[tgt_v.at[g0]], add=True)
                    g2 = jnp.minimum(g0 + 2, nch - 2)
                    ha = gath(g2, ra_v, sema)
                    hb.wait()
                    pltpu.sync_copy(rb_v, acc.at[tgt_v.at[g0 + 1]], add=True)
                    ha.wait()

                lax.fori_loop(0, nch // 2,
                              lambda t, _: (pbody(t * 2), 0)[1], 0)
            plsc.subcore_barrier()

            def fcp(u, _):
                st = sid * fpt + u * 64
                pltpu.sync_copy(acc.at[pl.ds(st, 64)],
                                out_h.at[pl.ds(lo + st, 64)])
                return 0

            lax.fori_loop(0, fpt // 64, fcp, 0)
            plsc.subcore_barrier()

    return k(src, dst, *nys)


def _scatter_counts(dst, n):
    return jnp.zeros((n,), F32).at[dst].add(1.0, mode="drop")


def _topk_select(rank, s, k, kp, dump):
    sel = rank < k
    idx = jnp.arange(rank.shape[0], dtype=jnp.int32)
    tgt = jnp.where(sel, rank, k)
    perm = jnp.full((kp + 1,), dump, jnp.int32).at[tgt].set(
        jnp.where(sel, idx, dump), mode="drop")
    vals = jnp.zeros((kp + 1,), F32).at[tgt].set(
        jnp.where(sel, s, 0.0), mode="drop")
    perm = perm.at[k].set(dump) if k < kp else perm
    vals = vals.at[k].set(0.0) if k < kp else vals
    return perm[:kp], vals[:kp]


def _build_restricted(inv, rows_idx, cols_idx, kp, n):
    a = inv[rows_idx]
    w = jnp.where(a >= 0, 1.0, 0.0).astype(jnp.bfloat16)
    a = jnp.where(a >= 0, a, kp)
    out = jnp.zeros((kp + 1, n), jnp.bfloat16).at[a, cols_idx].add(
        w, mode="drop")
    return out[:kp]


def kernel(x, edge_index, W0, b0, W1, b1, W2, b2, W3, b3, W4, b4, p0, p1):
    n0 = x.shape[0]
    k1 = int(math.ceil(0.5 * n0))
    k2 = int(math.ceil(0.5 * k1))
    P0, P1, P2 = _rup(n0, 512), _rup(k1, 512), _rup(k2, 512)
    D0 = x.shape[1]

    src = edge_index[0]
    dst = edge_index[1]
    E = src.shape[0]
    Ep = _rup(E, 2048)
    srcp = jnp.pad(src, (0, Ep - E))
    dstp = jnp.pad(dst, (0, Ep - E), constant_values=P0)

    xp = jnp.pad(x, ((0, P0 - n0), (0, 0)))
    p0b = jnp.broadcast_to(p0[:, None], (p0.shape[0], 128))
    p1b = jnp.broadcast_to(p1[:, None], (p1.shape[0], 128))
    b0r, b1r, b2r = b0[None, :], b1[None, :], b2[None, :]
    b3r, b4r = b3[None, :], b4[None, :]

    (cnt0b,) = _sc_edge_agg(srcp, dstp, [None], P0)
    y0 = _xw_apply(xp, W0, cnt0b)
    agg0a, agg0b = _sc_edge_agg(srcp, dstp, [y0[:, :128], y0[:, 128:]], P0)
    agg0 = jnp.concatenate([agg0a, agg0b], axis=1)
    x1 = _relu_combine(agg0, y0, cnt0b, b0r)
    res0 = x1

    sb0, st0 = _scores(x1, p0b, n0)
    rank0 = _ranks(sb0, st0, n0)[:, 0]
    perm0, vals0 = _sc_topk_select(rank0, sb0[:, 0], k1, P1, P0 - 1)
    x2g = _sc_gather_rows(x1, perm0)
    vals0b = jnp.broadcast_to(vals0[:, None], (P1, 128))

    inv0 = jnp.full((P0,), -1, jnp.int32).at[perm0].set(
        jnp.arange(P1, dtype=jnp.int32), mode="drop")
    inv0 = inv0.at[P0 - 1].set(-1)
    au = jnp.concatenate([src, perm0])
    av = jnp.concatenate([dst, perm0])
    Mr = _build_restricted(inv0, au, av, P1, P0)
    Mtr = _build_restricted(inv0, av, au, P1, P0)
    S1 = _mm_bt(Mr, Mtr, zero_diag=True, bm=1024, bn=1024, bk=512)
    S1T = _transpose(S1)
    cnt1b = _rowsum(S1T)

    y1 = _xw_apply(x2g, W1, cnt1b, scale=vals0b)
    x3 = _conv_apply(S1T, y1, cnt1b, b1r)
    res1 = x3

    sb1, st1 = _scores(x3, p1b, k1)
    rank1 = _ranks(sb1, st1, k1)[:, 0]
    perm1, vals1 = _sc_topk_select(rank1, sb1[:, 0], k2, P2, P1 - 1)
    vals1b = jnp.broadcast_to(vals1[:, None], (P2, 128))

    G2, H2, x4g = _sc_gather_multi([S1, S1T, x3], perm1)
    perm1b = jnp.broadcast_to(perm1[:, None], (P2, 128))
    S2 = _mm_bt(G2, H2, zero_diag=True, diagp=perm1b,
                bm=512, bn=512, bk=512)
    S2T = _transpose(S2)
    cnt2b = _rowsum(S2T)

    y2 = _xw_apply(x4g, W2, cnt2b, scale=vals1b)
    x5 = _conv_apply(S2T, y2, cnt2b, b2r)

    h1 = _sc_up_add(res1, x5, perm1)
    y3 = _xw_apply(h1, W3, cnt1b)
    x6 = _conv_apply(S1T, y3, cnt1b, b3r)

    h0 = _sc_up_add(res0, x6, perm0)
    y4 = _xw_apply(h0, W4, cnt0b)
    (agg4,) = _sc_edge_agg(srcp, dstp, [y4], P0)
    x7 = _relu_combine(agg4, y4, cnt0b, b4r)
    return x7[:n0]


def _relu_kernel(agg_ref, y_ref, cnt_ref, b_ref, o_ref):
    dinv = lax.rsqrt(cnt_ref[:, 0:1] + 1.0)
    bias = jnp.broadcast_to(b_ref[...], o_ref.shape)
    o_ref[...] = jnp.maximum(dinv * (agg_ref[...] + y_ref[...]) + bias, 0.0)


def _relu_combine(agg, y, cnt_b, bias, bm=512):
    M, F = y.shape
    bn = F
    bm = min(bm, M)
    return pl.pallas_call(
        _relu_kernel,
        grid=(M // bm, F // bn),
        in_specs=[
            pl.BlockSpec((bm, bn), lambda i, j: (i, j)),
            pl.BlockSpec((bm, bn), lambda i, j: (i, j)),
            pl.BlockSpec((bm, 128), lambda i, j: (i, 0)),
            pl.BlockSpec((1, bn), lambda i, j: (0, j)),
        ],
        out_specs=pl.BlockSpec((bm, bn), lambda i, j: (i, j)),
        out_shape=jax.ShapeDtypeStruct((M, F), F32),
    )(agg, y, cnt_b, bias)
